# Initial kernel scaffold; baseline (speedup 1.0000x reference)
import functools

import jax
import jax.numpy as jnp
from jax import lax
from jax.experimental import pallas as pl
from jax.experimental.pallas import tpu as pltpu
from jax.experimental.pallas import tpu_sc as plsc

_B = 320000
_D = 128
_RPB = 128
_NB = _B // _RPB
_NC = 2
_NS = 16
_NW = _NC * _NS
_NBF = _NB // _NW
_REM = _NB - _NBF * _NW

_mesh = plsc.VectorSubcoreMesh(core_axis_name="c", subcore_axis_name="s")


@functools.partial(
    pl.kernel,
    mesh=_mesh,
    out_type=jax.ShapeDtypeStruct((_B, _D), jnp.float32),
    scratch_types=[
        pltpu.VMEM((_NBF + 1, _RPB), jnp.int32),
        pltpu.VMEM((_RPB, _D), jnp.float32),
        pltpu.SemaphoreType.DMA,
    ],
)
def _gather(x_hbm, src_hbm, out_hbm, idx_v, rows_v, sem):
    wid = lax.axis_index("s") * _NC + lax.axis_index("c")
    b0 = _NBF * wid + jnp.minimum(wid, _REM)
    n_b = _NBF + jnp.where(wid < _REM, 1, 0)

    pltpu.sync_copy(src_hbm.at[pl.ds(b0, _NBF)], idx_v.at[pl.ds(0, _NBF)])

    @pl.when(wid < _REM)
    def _():
        pltpu.sync_copy(src_hbm.at[pl.ds(b0 + _NBF, 1)],
                        idx_v.at[pl.ds(_NBF, 1)])

    def body(i, carry):
        pltpu.async_copy(x_hbm.at[idx_v.at[i]], rows_v, sem).wait()
        pltpu.sync_copy(rows_v, out_hbm.at[pl.ds((b0 + i) * _RPB, _RPB)])
        return carry

    lax.fori_loop(0, n_b, body, 0)


def kernel(x, edge_index):
    src = edge_index[0].astype(jnp.int32).reshape(_NB, _RPB)
    messages = _gather(x, src)
    return (x, edge_index, messages)


# speedup vs baseline: 4.0014x; 4.0014x over previous
import functools

import jax
import jax.numpy as jnp
from jax import lax
from jax.experimental import pallas as pl
from jax.experimental.pallas import tpu as pltpu
from jax.experimental.pallas import tpu_sc as plsc

_B = 320000
_D = 128
_RPB = 128
_NB = _B // _RPB
_NC = 2
_NS = 16
_NW = _NC * _NS
_NBF = _NB // _NW
_REM = _NB - _NBF * _NW

_mesh = plsc.VectorSubcoreMesh(core_axis_name="c", subcore_axis_name="s")


@functools.partial(
    pl.kernel,
    mesh=_mesh,
    out_type=jax.ShapeDtypeStruct((_B, _D), jnp.float32),
    scratch_types=[
        pltpu.VMEM(((_NBF + 1) * _RPB,), jnp.int32),
        pltpu.VMEM((_RPB, _D), jnp.float32),
        pltpu.SemaphoreType.DMA,
    ],
)
def _gather(x_hbm, src_hbm, out_hbm, idx_v, rows_v, sem):
    wid = lax.axis_index("s") * _NC + lax.axis_index("c")
    b0 = _NBF * wid + jnp.minimum(wid, _REM)
    n_b = _NBF + jnp.where(wid < _REM, 1, 0)

    pltpu.sync_copy(src_hbm.at[pl.ds(b0 * _RPB, _NBF * _RPB)],
                    idx_v.at[pl.ds(0, _NBF * _RPB)])

    @pl.when(wid < _REM)
    def _():
        pltpu.sync_copy(src_hbm.at[pl.ds((b0 + _NBF) * _RPB, _RPB)],
                        idx_v.at[pl.ds(_NBF * _RPB, _RPB)])

    def body(i, carry):
        pltpu.async_copy(x_hbm.at[idx_v.at[pl.ds(i * _RPB, _RPB)]],
                         rows_v, sem).wait()
        pltpu.sync_copy(rows_v, out_hbm.at[pl.ds((b0 + i) * _RPB, _RPB)])
        return carry

    lax.fori_loop(0, n_b, body, 0)


def kernel(x, edge_index):
    src = edge_index[0].astype(jnp.int32)
    messages = _gather(x, src)
    return (x, edge_index, messages)


# speedup vs baseline: 5.3256x; 1.3309x over previous
import functools

import jax
import jax.numpy as jnp
from jax import lax
from jax.experimental import pallas as pl
from jax.experimental.pallas import tpu as pltpu
from jax.experimental.pallas import tpu_sc as plsc

_B = 320000
_D = 128
_RPB = 128
_NB = _B // _RPB
_NC = 2
_NS = 16
_NW = _NC * _NS
_NBF = _NB // _NW
_REM = _NB - _NBF * _NW
_K = 3
_G = _NBF // _K
_PAIRS = _G // 2

_mesh = plsc.VectorSubcoreMesh(core_axis_name="c", subcore_axis_name="s")


@functools.partial(
    pl.kernel,
    mesh=_mesh,
    out_type=jax.ShapeDtypeStruct((_B, _D), jnp.float32),
    scratch_types=[
        pltpu.VMEM(((_NBF + 1) * _RPB,), jnp.int32),
        pltpu.VMEM((2, _K, _RPB, _D), jnp.float32),
        pltpu.SemaphoreType.DMA,
        pltpu.SemaphoreType.DMA,
    ],
)
def _gather(x_hbm, src_hbm, out_hbm, idx_v, rows_v, gsem, wsem):
    wid = lax.axis_index("s") * _NC + lax.axis_index("c")
    b0 = _NBF * wid + jnp.minimum(wid, _REM)
    n_extra = jnp.where(wid < _REM, 1, 0)

    pltpu.sync_copy(src_hbm.at[pl.ds(b0 * _RPB, _NBF * _RPB)],
                    idx_v.at[pl.ds(0, _NBF * _RPB)])

    @pl.when(wid < _REM)
    def _():
        pltpu.sync_copy(src_hbm.at[pl.ds((b0 + _NBF) * _RPB, _RPB)],
                        idx_v.at[pl.ds(_NBF * _RPB, _RPB)])

    def fire_gather(g, p):
        for k in range(_K):
            pltpu.make_async_copy(
                x_hbm.at[idx_v.at[pl.ds((g * _K + k) * _RPB, _RPB)]],
                rows_v.at[p, k], gsem).start()

    def drain_gather(p):
        for k in range(_K):
            pltpu.make_async_copy(x_hbm.at[pl.ds(0, _RPB)],
                                  rows_v.at[p, k], gsem).wait()

    def fire_store(g, p):
        for k in range(_K):
            pltpu.make_async_copy(
                rows_v.at[p, k],
                out_hbm.at[pl.ds((b0 + g * _K + k) * _RPB, _RPB)],
                wsem).start()

    def drain_store(p):
        for k in range(_K):
            pltpu.make_async_copy(
                rows_v.at[p, k], out_hbm.at[pl.ds(0, _RPB)], wsem).wait()

    fire_gather(0, 0)

    def pair(i, carry):
        ga = 2 * i
        drain_gather(0)

        @pl.when(i > 0)
        def _():
            drain_store(1)

        fire_gather(ga + 1, 1)
        fire_store(ga, 0)
        drain_gather(1)
        drain_store(0)

        @pl.when(i < _PAIRS - 1)
        def _():
            fire_gather(ga + 2, 0)

        fire_store(ga + 1, 1)
        return carry

    lax.fori_loop(0, _PAIRS, pair, 0)
    drain_store(1)

    @pl.when(n_extra > 0)
    def _():
        pltpu.async_copy(
            x_hbm.at[idx_v.at[pl.ds(_NBF * _RPB, _RPB)]],
            rows_v.at[0, 0], gsem).wait()
        pltpu.sync_copy(rows_v.at[0, 0],
                        out_hbm.at[pl.ds((b0 + _NBF) * _RPB, _RPB)])


def kernel(x, edge_index):
    src = edge_index[0].astype(jnp.int32)
    messages = _gather(x, src)
    return (x, edge_index, messages)


# speedup vs baseline: 5.3284x; 1.0005x over previous
import functools

import jax
import jax.numpy as jnp
from jax import lax
from jax.experimental import pallas as pl
from jax.experimental.pallas import tpu as pltpu
from jax.experimental.pallas import tpu_sc as plsc

_B = 320000
_D = 128
_RPB = 128
_NB = _B // _RPB
_NC = 2
_NS = 16
_NW = _NC * _NS
_NBF = _NB // _NW
_REM = _NB - _NBF * _NW
_K = 3
_GR = _K * _RPB
_G = _NBF // _K
_PAIRS = _G // 2

_mesh = plsc.VectorSubcoreMesh(core_axis_name="c", subcore_axis_name="s")


@functools.partial(
    pl.kernel,
    mesh=_mesh,
    out_type=jax.ShapeDtypeStruct((_B, _D), jnp.float32),
    scratch_types=[
        pltpu.VMEM(((_NBF + 1) * _RPB,), jnp.int32),
        pltpu.VMEM((2, _GR, _D), jnp.float32),
        pltpu.SemaphoreType.DMA,
        pltpu.SemaphoreType.DMA,
    ],
)
def _gather(x_hbm, src_hbm, out_hbm, idx_v, rows_v, gsem, wsem):
    wid = lax.axis_index("s") * _NC + lax.axis_index("c")
    b0 = _NBF * wid + jnp.minimum(wid, _REM)
    r0 = b0 * _RPB

    pltpu.sync_copy(src_hbm.at[pl.ds(r0, _NBF * _RPB)],
                    idx_v.at[pl.ds(0, _NBF * _RPB)])

    @pl.when(wid < _REM)
    def _():
        pltpu.sync_copy(src_hbm.at[pl.ds(r0 + _NBF * _RPB, _RPB)],
                        idx_v.at[pl.ds(_NBF * _RPB, _RPB)])

    def fire_gather(g, p):
        pltpu.make_async_copy(
            x_hbm.at[idx_v.at[pl.ds(g * _GR, _GR)]],
            rows_v.at[p], gsem).start()

    def drain_gather(p):
        pltpu.make_async_copy(x_hbm.at[pl.ds(0, _GR)],
                              rows_v.at[p], gsem).wait()

    def fire_store(g, p):
        pltpu.make_async_copy(
            rows_v.at[p], out_hbm.at[pl.ds(r0 + g * _GR, _GR)],
            wsem).start()

    def drain_store(p):
        pltpu.make_async_copy(
            rows_v.at[p], out_hbm.at[pl.ds(0, _GR)], wsem).wait()

    fire_gather(0, 0)

    def pair(i, carry):
        ga = 2 * i
        drain_gather(0)

        @pl.when(i > 0)
        def _():
            drain_store(1)

        fire_gather(ga + 1, 1)
        fire_store(ga, 0)
        drain_gather(1)
        drain_store(0)

        @pl.when(i < _PAIRS - 1)
        def _():
            fire_gather(ga + 2, 0)

        fire_store(ga + 1, 1)
        return carry

    lax.fori_loop(0, _PAIRS, pair, 0)
    drain_store(1)

    @pl.when(wid < _REM)
    def _():
        pltpu.async_copy(
            x_hbm.at[idx_v.at[pl.ds(_NBF * _RPB, _RPB)]],
            rows_v.at[0, pl.ds(0, _RPB)], gsem).wait()
        pltpu.sync_copy(rows_v.at[0, pl.ds(0, _RPB)],
                        out_hbm.at[pl.ds(r0 + _NBF * _RPB, _RPB)])


def kernel(x, edge_index):
    src = edge_index[0].astype(jnp.int32)
    messages = _gather(x, src)
    return (x, edge_index, messages)


# speedup vs baseline: 5.5251x; 1.0369x over previous
import functools

import jax
import jax.numpy as jnp
from jax import lax
from jax.experimental import pallas as pl
from jax.experimental.pallas import tpu as pltpu
from jax.experimental.pallas import tpu_sc as plsc

_B = 320000
_D = 128
_RPB = 128
_NB = _B // _RPB
_NC = 2
_NS = 16
_NW = _NC * _NS
_NBF = _NB // _NW
_REM = _NB - _NBF * _NW
_K = 2
_GR = _K * _RPB
_G = _NBF // _K
_TRIPS = _G // 3

_mesh = plsc.VectorSubcoreMesh(core_axis_name="c", subcore_axis_name="s")


@functools.partial(
    pl.kernel,
    mesh=_mesh,
    out_type=jax.ShapeDtypeStruct((_B, _D), jnp.float32),
    scratch_types=[
        pltpu.VMEM(((_NBF + 1) * _RPB,), jnp.int32),
        pltpu.VMEM((3, _GR, _D), jnp.float32),
        pltpu.SemaphoreType.DMA,
        pltpu.SemaphoreType.DMA,
    ],
)
def _gather(x_hbm, src_hbm, out_hbm, idx_v, rows_v, gsem, wsem):
    wid = lax.axis_index("s") * _NC + lax.axis_index("c")
    b0 = _NBF * wid + jnp.minimum(wid, _REM)
    r0 = b0 * _RPB

    pltpu.sync_copy(src_hbm.at[pl.ds(r0, _NBF * _RPB)],
                    idx_v.at[pl.ds(0, _NBF * _RPB)])

    @pl.when(wid < _REM)
    def _():
        pltpu.sync_copy(src_hbm.at[pl.ds(r0 + _NBF * _RPB, _RPB)],
                        idx_v.at[pl.ds(_NBF * _RPB, _RPB)])

    def fire_gather(g, p):
        pltpu.make_async_copy(
            x_hbm.at[idx_v.at[pl.ds(g * _GR, _GR)]],
            rows_v.at[p], gsem).start()

    def drain_gather(p):
        pltpu.make_async_copy(x_hbm.at[pl.ds(0, _GR)],
                              rows_v.at[p], gsem).wait()

    def fire_store(g, p):
        pltpu.make_async_copy(
            rows_v.at[p], out_hbm.at[pl.ds(r0 + g * _GR, _GR)],
            wsem).start()

    def drain_store(p):
        pltpu.make_async_copy(
            rows_v.at[p], out_hbm.at[pl.ds(0, _GR)], wsem).wait()

    fire_gather(0, 0)
    fire_gather(1, 1)

    def ring(i, carry):
        g = 3 * i
        drain_gather(0)

        @pl.when(i > 0)
        def _():
            drain_store(2)

        fire_gather(g + 2, 2)
        fire_store(g, 0)
        drain_gather(1)
        drain_store(0)

        @pl.when(i < _TRIPS - 1)
        def _():
            fire_gather(g + 3, 0)

        fire_store(g + 1, 1)
        drain_gather(2)
        drain_store(1)

        @pl.when(i < _TRIPS - 1)
        def _():
            fire_gather(g + 4, 1)

        fire_store(g + 2, 2)
        return carry

    lax.fori_loop(0, _TRIPS, ring, 0)
    drain_store(2)

    @pl.when(wid < _REM)
    def _():
        pltpu.async_copy(
            x_hbm.at[idx_v.at[pl.ds(_NBF * _RPB, _RPB)]],
            rows_v.at[0, pl.ds(0, _RPB)], gsem).wait()
        pltpu.sync_copy(rows_v.at[0, pl.ds(0, _RPB)],
                        out_hbm.at[pl.ds(r0 + _NBF * _RPB, _RPB)])


def kernel(x, edge_index):
    src = edge_index[0].astype(jnp.int32)
    messages = _gather(x, src)
    return (x, edge_index, messages)


# speedup vs baseline: 6.0012x; 1.0862x over previous
import functools

import jax
import jax.numpy as jnp
from jax import lax
from jax.experimental import pallas as pl
from jax.experimental.pallas import tpu as pltpu
from jax.experimental.pallas import tpu_sc as plsc

_B = 320000
_D = 128
_RPB = 128
_NB = _B // _RPB
_NC = 2
_NS = 16
_NW = _NC * _NS
_NBF = _NB // _NW
_REM = _NB - _NBF * _NW
_K = 2
_GR = _K * _RPB
_G = _NBF // _K
_TRIPS = _G // 3

_mesh = plsc.VectorSubcoreMesh(core_axis_name="c", subcore_axis_name="s")


@functools.partial(
    pl.kernel,
    mesh=_mesh,
    out_type=jax.ShapeDtypeStruct((_B, _D), jnp.float32),
    scratch_types=[
        pltpu.VMEM(((_NBF + 1) * _RPB,), jnp.int32),
        pltpu.VMEM((3, _GR, _D), jnp.float32),
        pltpu.SemaphoreType.DMA,
        pltpu.SemaphoreType.DMA,
    ],
)
def _gather(x_hbm, src_hbm, out_hbm, idx_v, rows_v, gsem, wsem):
    wid = lax.axis_index("s") * _NC + lax.axis_index("c")
    b0 = _NBF * wid + jnp.minimum(wid, _REM)
    r0 = b0 * _RPB

    pltpu.sync_copy(src_hbm.at[0, pl.ds(r0, _NBF * _RPB)],
                    idx_v.at[pl.ds(0, _NBF * _RPB)])

    @pl.when(wid < _REM)
    def _():
        pltpu.sync_copy(src_hbm.at[0, pl.ds(r0 + _NBF * _RPB, _RPB)],
                        idx_v.at[pl.ds(_NBF * _RPB, _RPB)])

    def fire_gather(g, p):
        pltpu.make_async_copy(
            x_hbm.at[idx_v.at[pl.ds(g * _GR, _GR)]],
            rows_v.at[p], gsem).start()

    def drain_gather(p):
        pltpu.make_async_copy(x_hbm.at[pl.ds(0, _GR)],
                              rows_v.at[p], gsem).wait()

    def fire_store(g, p):
        pltpu.make_async_copy(
            rows_v.at[p], out_hbm.at[pl.ds(r0 + g * _GR, _GR)],
            wsem).start()

    def drain_store(p):
        pltpu.make_async_copy(
            rows_v.at[p], out_hbm.at[pl.ds(0, _GR)], wsem).wait()

    fire_gather(0, 0)
    fire_gather(1, 1)

    def ring(i, carry):
        g = 3 * i
        drain_gather(0)

        @pl.when(i > 0)
        def _():
            drain_store(2)

        fire_gather(g + 2, 2)
        fire_store(g, 0)
        drain_gather(1)
        drain_store(0)

        @pl.when(i < _TRIPS - 1)
        def _():
            fire_gather(g + 3, 0)

        fire_store(g + 1, 1)
        drain_gather(2)
        drain_store(1)

        @pl.when(i < _TRIPS - 1)
        def _():
            fire_gather(g + 4, 1)

        fire_store(g + 2, 2)
        return carry

    lax.fori_loop(0, _TRIPS, ring, 0)
    drain_store(2)

    @pl.when(wid < _REM)
    def _():
        pltpu.async_copy(
            x_hbm.at[idx_v.at[pl.ds(_NBF * _RPB, _RPB)]],
            rows_v.at[0, pl.ds(0, _RPB)], gsem).wait()
        pltpu.sync_copy(rows_v.at[0, pl.ds(0, _RPB)],
                        out_hbm.at[pl.ds(r0 + _NBF * _RPB, _RPB)])


def kernel(x, edge_index):
    messages = _gather(x, edge_index.astype(jnp.int32))
    return (x, edge_index, messages)
